# Initial kernel scaffold; baseline (speedup 1.0000x reference)
import jax
import jax.numpy as jnp
from jax.experimental import pallas as pl


def kernel(x, H, De, batch, y, weight_lap, W1, b1, W2, b2, W3, b3, Wp1, bp1, Wp2, bp2):
    raise NotImplementedError("write your pallas kernel here")


# speedup vs baseline: 441.7858x; 441.7858x over previous
import jax
import jax.numpy as jnp
from jax.experimental import pallas as pl
from jax.experimental.pallas import tpu as pltpu

B = 16
W = 128
D_IN = 128
H3 = 256
OUT = 16

_F32 = jnp.float32


def _fwd_kernel(H_ref, De_ref, x_ref, b_ref, wl_ref,
                W1_ref, b1_ref, W2_ref, b2_ref, W3_ref, b3_ref,
                Wp1_ref, bp1_ref, Wp2_ref, bp2_ref,
                L_ref, emb_ref, logp_ref,
                sums_ref, cnts_ref):
    g = pl.program_id(0)

    Hb = H_ref[0]
    Deb = De_ref[0]
    wl = jnp.abs(wl_ref[0, :])

    Hw = Hb * wl[None, :]
    ones_row = jnp.ones((1, W), dtype=_F32)
    rs_col = jnp.sum(Hw, axis=1, keepdims=True)
    rs_lane = jax.lax.dot_general(
        ones_row, Hw, (((1,), (1,)), ((), ())),
        preferred_element_type=_F32)
    dinv_col = jax.lax.rsqrt(rs_col)
    dinv_lane = jax.lax.rsqrt(rs_lane)

    M1 = jax.lax.dot_general(
        Deb, Hb, (((1,), (1,)), ((), ())),
        preferred_element_type=_F32)
    L0 = jnp.dot(Hw, M1, preferred_element_type=_F32)
    Lb = L0 * dinv_col * dinv_lane
    L_ref[0] = Lb

    ones_col = jnp.ones((W, 1), dtype=_F32)
    deg_lane = jax.lax.dot_general(
        ones_row, Lb, (((1,), (0,)), ((), ())),
        preferred_element_type=_F32)
    deg_col = jax.lax.dot_general(
        Lb, ones_col, (((0,), (0,)), ((), ())),
        preferred_element_type=_F32)
    dis_lane = jnp.where(deg_lane > 0, jax.lax.rsqrt(deg_lane), 0.0)
    dis_col = jnp.where(deg_col > 0, jax.lax.rsqrt(deg_col), 0.0)
    Ln = Lb * dis_col * dis_lane

    def conv(h, Wm_ref, bias_ref):
        p = jnp.dot(h, Wm_ref[...], preferred_element_type=_F32)
        out = jax.lax.dot_general(
            Ln, p, (((0,), (0,)), ((), ())),
            preferred_element_type=_F32)
        return out + bias_ref[0, :][None, :]

    h1 = conv(x_ref[...], W1_ref, b1_ref)
    h2 = conv(jax.nn.relu(h1), W2_ref, b2_ref)
    h3 = conv(jax.nn.relu(h2), W3_ref, b3_ref)
    emb_ref[0] = h3
    r3 = jax.nn.relu(h3)

    bv = b_ref[0]
    row_ids = jax.lax.broadcasted_iota(jnp.int32, (B, W), 0)
    ohT = (jnp.broadcast_to(bv, (B, W)) == row_ids).astype(_F32)

    @pl.when(g == 0)
    def _init():
        sums_ref[...] = jnp.zeros_like(sums_ref)
        cnts_ref[...] = jnp.zeros_like(cnts_ref)

    ones_feat = jnp.ones((W, H3), dtype=_F32)
    sums_ref[...] += jnp.dot(ohT, r3, preferred_element_type=_F32)
    cnts_ref[...] += jnp.dot(ohT, ones_feat, preferred_element_type=_F32)

    @pl.when(g == B - 1)
    def _head():
        pooled = sums_ref[...] / jnp.maximum(cnts_ref[...], 1.0)
        o = jnp.dot(pooled, Wp1_ref[...],
                    preferred_element_type=_F32) + bp1_ref[0, :][None, :]
        o = jnp.dot(o, Wp2_ref[...],
                    preferred_element_type=_F32) + bp2_ref[0, :][None, :]
        m = jnp.max(o, axis=1, keepdims=True)
        lse = jnp.log(jnp.sum(jnp.exp(o - m), axis=1, keepdims=True)) + m
        logp_ref[...] = o - lse


def kernel(x, H, De, batch, y, weight_lap, W1, b1, W2, b2, W3, b3,
           Wp1, bp1, Wp2, bp2):
    del y
    batch3 = batch.astype(jnp.int32).reshape(B, 1, W)
    wl2 = weight_lap.reshape(1, W)

    def c(shape):
        return pl.BlockSpec(shape, lambda g: (0,) * len(shape))

    in_specs = [
            pl.BlockSpec((1, W, W), lambda g: (g, 0, 0)),
            pl.BlockSpec((1, W, W), lambda g: (g, 0, 0)),
            pl.BlockSpec((W, D_IN), lambda g: (g, 0)),
            pl.BlockSpec((1, 1, W), lambda g: (g, 0, 0)),
            c((1, W)),
            c((D_IN, H3)), c((1, H3)),
            c((H3, H3)), c((1, H3)),
            c((H3, H3)), c((1, H3)),
            c((H3, H3)), c((1, H3)),
            c((H3, OUT)), c((1, OUT)),
    ]
    out_specs = [
        pl.BlockSpec((1, W, W), lambda g: (g, 0, 0)),
        pl.BlockSpec((1, W, H3), lambda g: (g, 0, 0)),
        pl.BlockSpec((B, OUT), lambda g: (0, 0)),
    ]

    L, emb, logp = pl.pallas_call(
        _fwd_kernel,
        grid=(B,),
        in_specs=in_specs,
        out_specs=out_specs,
        out_shape=[
            jax.ShapeDtypeStruct((B, W, W), _F32),
            jax.ShapeDtypeStruct((B, W, H3), _F32),
            jax.ShapeDtypeStruct((B, OUT), _F32),
        ],
        scratch_shapes=[
            pltpu.VMEM((B, H3), _F32),
            pltpu.VMEM((B, H3), _F32),
        ],
    )(H, De, x, batch3, wl2,
      W1, b1.reshape(1, H3), W2, b2.reshape(1, H3), W3, b3.reshape(1, H3),
      Wp1, bp1.reshape(1, H3), Wp2, bp2.reshape(1, OUT))

    return (emb.reshape(B * W, H3), logp, weight_lap, L)


# speedup vs baseline: 488.3111x; 1.1053x over previous
import jax
import jax.numpy as jnp
from jax.experimental import pallas as pl
from jax.experimental.pallas import tpu as pltpu

B = 16
W = 128
D_IN = 128
H3 = 256
OUT = 16

_F32 = jnp.float32


GPB = 4
STEPS = B // GPB


def _graph_chain(Hb, Deb, wl, x_blk, W1_ref, b1_ref, W2_ref, b2_ref,
                 W3_ref, b3_ref):
    Hw = Hb * wl[None, :]
    ones_row = jnp.ones((1, W), dtype=_F32)
    rs_col = jnp.sum(Hw, axis=1, keepdims=True)
    rs_lane = jax.lax.dot_general(
        ones_row, Hw, (((1,), (1,)), ((), ())),
        preferred_element_type=_F32)
    dinv_col = jax.lax.rsqrt(rs_col)
    dinv_lane = jax.lax.rsqrt(rs_lane)

    M1 = jax.lax.dot_general(
        Deb, Hb, (((1,), (1,)), ((), ())),
        preferred_element_type=_F32)
    L0 = jnp.dot(Hw, M1, preferred_element_type=_F32)
    Lb = L0 * dinv_col * dinv_lane

    ones_col = jnp.ones((W, 1), dtype=_F32)
    deg_lane = jax.lax.dot_general(
        ones_row, Lb, (((1,), (0,)), ((), ())),
        preferred_element_type=_F32)
    deg_col = jax.lax.dot_general(
        Lb, ones_col, (((0,), (0,)), ((), ())),
        preferred_element_type=_F32)
    dis_lane = jnp.where(deg_lane > 0, jax.lax.rsqrt(deg_lane), 0.0)
    dis_col = jnp.where(deg_col > 0, jax.lax.rsqrt(deg_col), 0.0)
    Ln = Lb * dis_col * dis_lane

    def conv(h, Wm_ref, bias_ref):
        p = jnp.dot(h, Wm_ref[...], preferred_element_type=_F32)
        out = jax.lax.dot_general(
            Ln, p, (((0,), (0,)), ((), ())),
            preferred_element_type=_F32)
        return out + bias_ref[0, :][None, :]

    h1 = conv(x_blk, W1_ref, b1_ref)
    h2 = conv(jax.nn.relu(h1), W2_ref, b2_ref)
    h3 = conv(jax.nn.relu(h2), W3_ref, b3_ref)
    return Lb, h3


def _fwd_kernel(H_ref, De_ref, x_ref, b_ref, wl_ref,
                W1_ref, b1_ref, W2_ref, b2_ref, W3_ref, b3_ref,
                Wp1_ref, bp1_ref, Wp2_ref, bp2_ref,
                L_ref, emb_ref, logp_ref,
                sums_ref, cnts_ref):
    g = pl.program_id(0)
    wl = jnp.abs(wl_ref[0, :])

    @pl.when(g == 0)
    def _init():
        sums_ref[...] = jnp.zeros_like(sums_ref)
        cnts_ref[...] = jnp.zeros_like(cnts_ref)

    row_ids = jax.lax.broadcasted_iota(jnp.int32, (B, W), 0)
    ones_feat = jnp.ones((W, H3), dtype=_F32)
    sums_acc = jnp.zeros((B, H3), _F32)
    cnts_acc = jnp.zeros((B, H3), _F32)
    for k in range(GPB):
        Lb, h3 = _graph_chain(
            H_ref[k], De_ref[k], wl, x_ref[k],
            W1_ref, b1_ref, W2_ref, b2_ref, W3_ref, b3_ref)
        L_ref[k] = Lb
        emb_ref[k] = h3
        r3 = jax.nn.relu(h3)
        bv = b_ref[0, k]
        ohT = (bv[None, :] == row_ids).astype(_F32)
        sums_acc += jnp.dot(ohT, r3, preferred_element_type=_F32)
        cnts_acc += jnp.dot(ohT, ones_feat, preferred_element_type=_F32)

    sums_ref[...] += sums_acc
    cnts_ref[...] += cnts_acc

    @pl.when(g == STEPS - 1)
    def _head():
        pooled = sums_ref[...] / jnp.maximum(cnts_ref[...], 1.0)
        o = jnp.dot(pooled, Wp1_ref[...],
                    preferred_element_type=_F32) + bp1_ref[0, :][None, :]
        o = jnp.dot(o, Wp2_ref[...],
                    preferred_element_type=_F32) + bp2_ref[0, :][None, :]
        m = jnp.max(o, axis=1, keepdims=True)
        lse = jnp.log(jnp.sum(jnp.exp(o - m), axis=1, keepdims=True)) + m
        logp_ref[...] = o - lse


def kernel(x, H, De, batch, y, weight_lap, W1, b1, W2, b2, W3, b3,
           Wp1, bp1, Wp2, bp2):
    del y
    batch3 = batch.astype(jnp.int32).reshape(STEPS, GPB, W)
    wl2 = weight_lap.reshape(1, W)
    x3 = x.reshape(B, W, D_IN)

    def c(shape):
        return pl.BlockSpec(shape, lambda g: (0,) * len(shape))

    in_specs = [
            pl.BlockSpec((GPB, W, W), lambda g: (g, 0, 0)),
            pl.BlockSpec((GPB, W, W), lambda g: (g, 0, 0)),
            pl.BlockSpec((GPB, W, D_IN), lambda g: (g, 0, 0)),
            pl.BlockSpec((1, GPB, W), lambda g: (g, 0, 0)),
            c((1, W)),
            c((D_IN, H3)), c((1, H3)),
            c((H3, H3)), c((1, H3)),
            c((H3, H3)), c((1, H3)),
            c((H3, H3)), c((1, H3)),
            c((H3, OUT)), c((1, OUT)),
    ]
    out_specs = [
        pl.BlockSpec((GPB, W, W), lambda g: (g, 0, 0)),
        pl.BlockSpec((GPB, W, H3), lambda g: (g, 0, 0)),
        pl.BlockSpec((B, OUT), lambda g: (0, 0)),
    ]

    L, emb, logp = pl.pallas_call(
        _fwd_kernel,
        grid=(STEPS,),
        in_specs=in_specs,
        out_specs=out_specs,
        out_shape=[
            jax.ShapeDtypeStruct((B, W, W), _F32),
            jax.ShapeDtypeStruct((B, W, H3), _F32),
            jax.ShapeDtypeStruct((B, OUT), _F32),
        ],
        scratch_shapes=[
            pltpu.VMEM((B, H3), _F32),
            pltpu.VMEM((B, H3), _F32),
        ],
    )(H, De, x3, batch3, wl2,
      W1, b1.reshape(1, H3), W2, b2.reshape(1, H3), W3, b3.reshape(1, H3),
      Wp1, bp1.reshape(1, H3), Wp2, bp2.reshape(1, OUT))

    return (emb.reshape(B * W, H3), logp, weight_lap, L)


# speedup vs baseline: 1385.8236x; 2.8380x over previous
import jax
import jax.numpy as jnp
from jax.experimental import pallas as pl
from jax.experimental.pallas import tpu as pltpu

B = 16
W = 128
D_IN = 128
H3 = 256
OUT = 16

_F32 = jnp.float32


GPB = 16
STEPS = B // GPB


def _dot(a, b):
    return jnp.dot(a, b, preferred_element_type=_F32)


def _dotT(a, b, dims):
    return jax.lax.dot_general(a, b, (dims, ((), ())),
                               preferred_element_type=_F32)


def _fwd_kernel(H_ref, De_ref, x_ref, b_ref, wl_ref,
                W1_ref, b1_ref, W2_ref, b2_ref, W3_ref, b3_ref,
                Wp1_ref, bp1_ref, Wp2_ref, bp2_ref,
                L_ref, emb_ref, logp_ref,
                sums_ref, cnts_ref):
    g = pl.program_id(0)
    wl = jnp.abs(wl_ref[0, :])

    @pl.when(g == 0)
    def _init():
        sums_ref[...] = jnp.zeros_like(sums_ref)
        cnts_ref[...] = jnp.zeros_like(cnts_ref)

    R = range(GPB)
    Hb = [H_ref[k] for k in R]
    Deb = [De_ref[k] for k in R]
    Hw = [Hb[k] * wl[None, :] for k in R]
    dinv = [jax.lax.rsqrt(jnp.sum(Hw[k], axis=1, keepdims=True))
            for k in R]
    Hs = [Hb[k] * dinv[k] for k in R]
    M1 = [_dotT(Deb[k], Hs[k], (((1,), (1,)))) for k in R]
    p1 = [_dot(x_ref[k], W1_ref[...]) for k in R]
    Lb = [_dot(Hw[k], M1[k]) * dinv[k] for k in R]
    for k in R:
        L_ref[k] = Lb[k]

    ones_col = jnp.ones((W, 1), dtype=_F32)
    deg = [_dotT(Lb[k], ones_col, (((0,), (0,)))) for k in R]
    dis = [jnp.where(deg[k] > 0, jax.lax.rsqrt(deg[k]), 0.0) for k in R]

    def agg(p, k, bias_ref):
        out = _dotT(Lb[k], dis[k] * p, (((0,), (0,)))) * dis[k]
        return out + bias_ref[0, :][None, :]

    h1 = [agg(p1[k], k, b1_ref) for k in R]
    p2 = [_dot(jax.nn.relu(h1[k]), W2_ref[...]) for k in R]
    h2 = [agg(p2[k], k, b2_ref) for k in R]
    p3 = [_dot(jax.nn.relu(h2[k]), W3_ref[...]) for k in R]
    h3 = [agg(p3[k], k, b3_ref) for k in R]
    for k in R:
        emb_ref[k] = h3[k]

    row_ids = jax.lax.broadcasted_iota(jnp.int32, (B, W), 0)
    ones_feat = jnp.ones((W, H3), dtype=_F32)
    ohT = [(b_ref[0, k][None, :] == row_ids).astype(_F32) for k in R]
    sums_acc = sum(_dot(ohT[k], jax.nn.relu(h3[k])) for k in R)
    cnts_acc = sum(_dot(ohT[k], ones_feat) for k in R)
    sums_ref[...] += sums_acc
    cnts_ref[...] += cnts_acc

    @pl.when(g == STEPS - 1)
    def _head():
        pooled = sums_ref[...] / jnp.maximum(cnts_ref[...], 1.0)
        o = jnp.dot(pooled, Wp1_ref[...],
                    preferred_element_type=_F32) + bp1_ref[0, :][None, :]
        o = jnp.dot(o, Wp2_ref[...],
                    preferred_element_type=_F32) + bp2_ref[0, :][None, :]
        m = jnp.max(o, axis=1, keepdims=True)
        lse = jnp.log(jnp.sum(jnp.exp(o - m), axis=1, keepdims=True)) + m
        logp_ref[...] = o - lse


def kernel(x, H, De, batch, y, weight_lap, W1, b1, W2, b2, W3, b3,
           Wp1, bp1, Wp2, bp2):
    del y
    batch3 = batch.astype(jnp.int32).reshape(STEPS, GPB, W)
    wl2 = weight_lap.reshape(1, W)
    x3 = x.reshape(B, W, D_IN)

    def c(shape):
        return pl.BlockSpec(shape, lambda g: (0,) * len(shape))

    in_specs = [
            pl.BlockSpec((GPB, W, W), lambda g: (g, 0, 0)),
            pl.BlockSpec((GPB, W, W), lambda g: (g, 0, 0)),
            pl.BlockSpec((GPB, W, D_IN), lambda g: (g, 0, 0)),
            pl.BlockSpec((1, GPB, W), lambda g: (g, 0, 0)),
            c((1, W)),
            c((D_IN, H3)), c((1, H3)),
            c((H3, H3)), c((1, H3)),
            c((H3, H3)), c((1, H3)),
            c((H3, H3)), c((1, H3)),
            c((H3, OUT)), c((1, OUT)),
    ]
    out_specs = [
        pl.BlockSpec((GPB, W, W), lambda g: (g, 0, 0)),
        pl.BlockSpec((GPB, W, H3), lambda g: (g, 0, 0)),
        pl.BlockSpec((B, OUT), lambda g: (0, 0)),
    ]

    L, emb, logp = pl.pallas_call(
        _fwd_kernel,
        grid=(STEPS,),
        in_specs=in_specs,
        out_specs=out_specs,
        out_shape=[
            jax.ShapeDtypeStruct((B, W, W), _F32),
            jax.ShapeDtypeStruct((B, W, H3), _F32),
            jax.ShapeDtypeStruct((B, OUT), _F32),
        ],
        scratch_shapes=[
            pltpu.VMEM((B, H3), _F32),
            pltpu.VMEM((B, H3), _F32),
        ],
    )(H, De, x3, batch3, wl2,
      W1, b1.reshape(1, H3), W2, b2.reshape(1, H3), W3, b3.reshape(1, H3),
      Wp1, bp1.reshape(1, H3), Wp2, bp2.reshape(1, OUT))

    return (emb.reshape(B * W, H3), logp, weight_lap, L)
